# Initial kernel scaffold; baseline (speedup 1.0000x reference)
import jax
import jax.numpy as jnp
from jax.experimental import pallas as pl


def kernel(c0, c1, c2, c3, c4, c5, c6, content_queries, scorer_w, scorer_b, anchor_w):
    raise NotImplementedError("write your pallas kernel here")


# speedup vs baseline: 2.0339x; 2.0339x over previous
import jax
import jax.numpy as jnp
from jax.experimental import pallas as pl
from jax.experimental.pallas import tpu as pltpu

_B, _T, _D, _K, _NS = 2, 4096, 1024, 7, 3
_BT = 256


def _fused_body(c0, c1, c2, c3, c4, c5, c6, w, wt, cq, out):
    cands = (c0, c1, c2, c3, c4, c5, c6)
    wv = w[:].astype(jnp.bfloat16).astype(jnp.float32)
    s = [
        jnp.sum(
            c[:].astype(jnp.bfloat16).astype(jnp.float32) * wv,
            axis=1,
            keepdims=True,
        )
        for c in cands
    ]

    neg = jnp.float32(-jnp.inf)
    slot_idx = []
    for _slot in range(_NS):
        m = s[0]
        am = jnp.zeros(m.shape, jnp.int32)
        for k in range(1, _K):
            gt = s[k] > m
            m = jnp.where(gt, s[k], m)
            am = jnp.where(gt, k, am)
        slot_idx.append(am)
        s = [jnp.where(am == k, neg, s[k]) for k in range(_K)]

    wtb = wt[:]
    ys = []
    for slot in range(_NS):
        am = slot_idx[slot]
        g = jnp.where(am == 0, cands[0][:], jnp.float32(0.0))
        for k in range(1, _K):
            g = jnp.where(am == k, cands[k][:], g)
        y = jax.lax.dot(
            g.astype(jnp.bfloat16), wtb, preferred_element_type=jnp.float32
        )
        ys.append(y + cq[slot])
    out[:] = jnp.stack(ys, axis=1)


def kernel(c0, c1, c2, c3, c4, c5, c6, content_queries, scorer_w, scorer_b, anchor_w):
    del scorer_b
    bt_total = _B * _T
    cands = [c.reshape(bt_total, _D) for c in (c0, c1, c2, c3, c4, c5, c6)]
    wt = anchor_w.T.astype(jnp.bfloat16)
    cq = content_queries.reshape(_NS, 1, _D)

    cand_spec = pl.BlockSpec((_BT, _D), lambda i: (i, 0))
    out = pl.pallas_call(
        _fused_body,
        grid=(bt_total // _BT,),
        in_specs=[cand_spec] * _K
        + [
            pl.BlockSpec((1, _D), lambda i: (0, 0)),
            pl.BlockSpec((_D, _D), lambda i: (0, 0)),
            pl.BlockSpec((_NS, 1, _D), lambda i: (0, 0, 0)),
        ],
        out_specs=pl.BlockSpec((_BT, _NS, _D), lambda i: (i, 0, 0)),
        out_shape=jax.ShapeDtypeStruct((bt_total, _NS, _D), jnp.float32),
        compiler_params=pltpu.CompilerParams(
            dimension_semantics=("parallel",),
        ),
    )(*cands, scorer_w, wt, cq)
    return out


# speedup vs baseline: 2.5086x; 1.2334x over previous
import jax
import jax.numpy as jnp
from jax.experimental import pallas as pl
from jax.experimental.pallas import tpu as pltpu

_B, _T, _D, _K, _NS = 2, 4096, 1024, 7, 3
_BT = 256


def _fused_body(c0, c1, c2, c3, c4, c5, c6, wb, wt, cq, out):
    cb = [c[:].astype(jnp.bfloat16) for c in (c0, c1, c2, c3, c4, c5, c6)]
    wbv = wb[:]

    s = [
        jax.lax.dot(c, wbv, preferred_element_type=jnp.float32) for c in cb
    ]

    neg = jnp.float32(-jnp.inf)
    slot_idx = []
    for _slot in range(_NS):
        m = s[0]
        am = jnp.zeros(m.shape, jnp.int32)
        for k in range(1, _K):
            gt = s[k] > m
            m = jnp.where(gt, s[k], m)
            am = jnp.where(gt, k, am)
        slot_idx.append(am)
        s = [jnp.where(am == k, neg, s[k]) for k in range(_K)]

    wtb = wt[:]
    for slot in range(_NS):
        am = slot_idx[slot]
        g = cb[0]
        for k in range(1, _K):
            g = jnp.where(am == k, cb[k], g)
        y = jax.lax.dot(g, wtb, preferred_element_type=jnp.float32)
        out[:, slot, :] = y + cq[slot]


def kernel(c0, c1, c2, c3, c4, c5, c6, content_queries, scorer_w, scorer_b, anchor_w):
    del scorer_b
    bt_total = _B * _T
    cands = [c.reshape(bt_total, _D) for c in (c0, c1, c2, c3, c4, c5, c6)]
    wb = scorer_w.reshape(_D, 1).astype(jnp.bfloat16)
    wt = anchor_w.T.astype(jnp.bfloat16)
    cq = content_queries.reshape(_NS, 1, _D)

    cand_spec = pl.BlockSpec((_BT, _D), lambda i: (i, 0))
    out = pl.pallas_call(
        _fused_body,
        grid=(bt_total // _BT,),
        in_specs=[cand_spec] * _K
        + [
            pl.BlockSpec((_D, 1), lambda i: (0, 0)),
            pl.BlockSpec((_D, _D), lambda i: (0, 0)),
            pl.BlockSpec((_NS, 1, _D), lambda i: (0, 0, 0)),
        ],
        out_specs=pl.BlockSpec((_BT, _NS, _D), lambda i: (i, 0, 0)),
        out_shape=jax.ShapeDtypeStruct((bt_total, _NS, _D), jnp.float32),
        compiler_params=pltpu.CompilerParams(
            dimension_semantics=("parallel",),
        ),
    )(*cands, wb, wt, cq)
    return out


# speedup vs baseline: 4.1426x; 1.6514x over previous
import jax
import jax.numpy as jnp
from jax.experimental import pallas as pl
from jax.experimental.pallas import tpu as pltpu

_B, _T, _D, _K, _NS = 2, 4096, 1024, 7, 3
_BT = 256


def _fused_body(c0, c1, c2, c3, c4, c5, c6, wb, wt, cq, out):
    cb = [c[:].astype(jnp.bfloat16) for c in (c0, c1, c2, c3, c4, c5, c6)]
    wbv = wb[:]

    s = [
        jax.lax.dot(c, wbv, preferred_element_type=jnp.float32) for c in cb
    ]

    neg = jnp.float32(-jnp.inf)
    slot_idx = []
    for _slot in range(_NS):
        m = s[0]
        am = jnp.zeros(m.shape, jnp.int32)
        for k in range(1, _K):
            gt = s[k] > m
            m = jnp.where(gt, s[k], m)
            am = jnp.where(gt, k, am)
        slot_idx.append(am)
        s = [jnp.where(am == k, neg, s[k]) for k in range(_K)]

    wtb = wt[:]
    for slot in range(_NS):
        am = slot_idx[slot]
        g = cb[0]
        for k in range(1, _K):
            g = jnp.where(am == k, cb[k], g)
        y = jax.lax.dot(g, wtb, preferred_element_type=jnp.float32)
        out[slot] = y + cq[slot]


def kernel(c0, c1, c2, c3, c4, c5, c6, content_queries, scorer_w, scorer_b, anchor_w):
    del scorer_b
    bt_total = _B * _T
    cands = [c.reshape(bt_total, _D) for c in (c0, c1, c2, c3, c4, c5, c6)]
    wb = scorer_w.reshape(_D, 1).astype(jnp.bfloat16)
    wt = anchor_w.T.astype(jnp.bfloat16)
    cq = content_queries.reshape(_NS, 1, _D)

    cand_spec = pl.BlockSpec((_BT, _D), lambda i: (i, 0))
    out = pl.pallas_call(
        _fused_body,
        grid=(bt_total // _BT,),
        in_specs=[cand_spec] * _K
        + [
            pl.BlockSpec((_D, 1), lambda i: (0, 0)),
            pl.BlockSpec((_D, _D), lambda i: (0, 0)),
            pl.BlockSpec((_NS, 1, _D), lambda i: (0, 0, 0)),
        ],
        out_specs=pl.BlockSpec((_NS, _BT, _D), lambda i: (0, i, 0)),
        out_shape=jax.ShapeDtypeStruct((_NS, bt_total, _D), jnp.float32),
        compiler_params=pltpu.CompilerParams(
            dimension_semantics=("parallel",),
        ),
    )(*cands, wb, wt, cq)
    return out.transpose(1, 0, 2)


# speedup vs baseline: 4.4420x; 1.0723x over previous
import jax
import jax.numpy as jnp
from jax.experimental import pallas as pl
from jax.experimental.pallas import tpu as pltpu

_B, _T, _D, _K, _NS = 2, 4096, 1024, 7, 3
_BT = 512


def _fused_body(c0, c1, c2, c3, c4, c5, c6, wb, wt, cq, out):
    cb = [c[:].astype(jnp.bfloat16) for c in (c0, c1, c2, c3, c4, c5, c6)]
    wbv = wb[:]

    s = [
        jax.lax.dot(c, wbv, preferred_element_type=jnp.float32) for c in cb
    ]

    neg = jnp.float32(-jnp.inf)
    slot_idx = []
    for _slot in range(_NS):
        m = s[0]
        am = jnp.zeros(m.shape, jnp.int32)
        for k in range(1, _K):
            gt = s[k] > m
            m = jnp.where(gt, s[k], m)
            am = jnp.where(gt, k, am)
        slot_idx.append(am)
        s = [jnp.where(am == k, neg, s[k]) for k in range(_K)]

    wtb = wt[:]
    for slot in range(_NS):
        am = slot_idx[slot]
        g = cb[0]
        for k in range(1, _K):
            g = jnp.where(am == k, cb[k], g)
        y = jax.lax.dot(g, wtb, preferred_element_type=jnp.float32)
        out[slot] = y + cq[slot]


def kernel(c0, c1, c2, c3, c4, c5, c6, content_queries, scorer_w, scorer_b, anchor_w):
    del scorer_b
    bt_total = _B * _T
    cands = [c.reshape(bt_total, _D) for c in (c0, c1, c2, c3, c4, c5, c6)]
    wb = scorer_w.reshape(_D, 1).astype(jnp.bfloat16)
    wt = anchor_w.T.astype(jnp.bfloat16)
    cq = content_queries.reshape(_NS, 1, _D)

    cand_spec = pl.BlockSpec((_BT, _D), lambda i: (i, 0))
    out = pl.pallas_call(
        _fused_body,
        grid=(bt_total // _BT,),
        in_specs=[cand_spec] * _K
        + [
            pl.BlockSpec((_D, 1), lambda i: (0, 0)),
            pl.BlockSpec((_D, _D), lambda i: (0, 0)),
            pl.BlockSpec((_NS, 1, _D), lambda i: (0, 0, 0)),
        ],
        out_specs=pl.BlockSpec((_NS, _BT, _D), lambda i: (0, i, 0)),
        out_shape=jax.ShapeDtypeStruct((_NS, bt_total, _D), jnp.float32),
        compiler_params=pltpu.CompilerParams(
            dimension_semantics=("parallel",),
        ),
    )(*cands, wb, wt, cq)
    return out.transpose(1, 0, 2)
